# baseline (speedup 1.0000x reference)
import functools

import jax
import jax.numpy as jnp
from jax.experimental import pallas as pl

BATCH = 16384
FIELD_DIM = 1000
EMBED_DIM = 64
BLOCK_M = 4096


def _embed_block(idx_ref, wt_ref, out_ref):
    mask = (idx_ref[...] != 0).astype(jnp.float32)
    out_ref[...] = jnp.dot(wt_ref[...], mask,
                           preferred_element_type=jnp.float32)


@functools.partial(jax.jit, static_argnames=())
def kernel(indices, W):
    idx_t = indices.T
    w_t = W.T
    out_t = pl.pallas_call(
        _embed_block,
        grid=(BATCH // BLOCK_M,),
        in_specs=[
            pl.BlockSpec((FIELD_DIM, BLOCK_M), lambda i: (0, i)),
            pl.BlockSpec((EMBED_DIM, FIELD_DIM), lambda i: (0, 0)),
        ],
        out_specs=pl.BlockSpec((EMBED_DIM, BLOCK_M), lambda i: (0, i)),
        out_shape=jax.ShapeDtypeStruct((EMBED_DIM, BATCH), jnp.float32),
    )(idx_t, w_t)
    return out_t.T


# speedup vs baseline: 1.0494x; 1.0494x over previous
import functools

import jax
import jax.numpy as jnp
from jax.experimental import pallas as pl

BATCH = 16384
FIELD_DIM = 1000
EMBED_DIM = 64
BLOCK_M = 2048
NSPLIT = 2
SUB_M = BLOCK_M // NSPLIT


def _embed_block(*refs):
    idx_refs = refs[:NSPLIT]
    wt_ref = refs[NSPLIT]
    out_ref = refs[NSPLIT + 1]
    wt = wt_ref[...]
    for k in range(NSPLIT):
        mask = (idx_refs[k][...] != 0).astype(jnp.float32)
        out_ref[:, k * SUB_M:(k + 1) * SUB_M] = jnp.dot(
            wt, mask, preferred_element_type=jnp.float32)


@functools.partial(jax.jit, static_argnames=())
def kernel(indices, W):
    idx_t = indices.T
    w_t = W.T

    def idx_spec(k):
        return pl.BlockSpec((FIELD_DIM, SUB_M),
                            lambda i, k=k: (0, i * NSPLIT + k))

    out_t = pl.pallas_call(
        _embed_block,
        grid=(BATCH // BLOCK_M,),
        in_specs=[idx_spec(k) for k in range(NSPLIT)] + [
            pl.BlockSpec((EMBED_DIM, FIELD_DIM), lambda i: (0, 0)),
        ],
        out_specs=pl.BlockSpec((EMBED_DIM, BLOCK_M), lambda i: (0, i)),
        out_shape=jax.ShapeDtypeStruct((EMBED_DIM, BATCH), jnp.float32),
    )(*([idx_t] * NSPLIT + [w_t]))
    return out_t.T
